# baseline (speedup 1.0000x reference)
import numpy as np

import jax
import jax.numpy as jnp
from jax import lax
from jax.experimental import pallas as pl
from jax.experimental.pallas import tpu as pltpu

_LANES = 512
_CHUNK = 128


def _phys_map(nb):
    pref = [6, 5, 4, 3] + list(range(7, nb - 1)) + [2, 1, 0]
    pref = [p for p in pref if p <= nb - 2]
    pi = {a: pref[a] for a in range(nb - 1)}
    pi[nb - 1] = nb - 1
    return pi


def _plan(nb):
    pi = _phys_map(nb)
    stages = []
    for b in range(1, nb):
        m = 2 ** pi[b]
        for a in range(b - 1, -1, -1):
            stages.append((m, 2 ** pi[a]))
    return stages


def _iota(n):
    return jax.lax.broadcasted_iota(jnp.int32, (n, 1), 0)


def _dirsel(m, desc, rows, hi, lo):
    if desc is not None:
        return (hi, lo) if desc else (lo, hi)
    if m == 0:
        return hi, lo
    d = (_iota(hi.shape[0]) & m) == 0
    return jnp.where(d, hi, lo), jnp.where(d, lo, hi)


def _cmpx_chunk(v, m, d, desc):
    if d >= 8:
        pieces = [None] * (2 * (v.shape[0] // (2 * d)))
        for g in range(v.shape[0] // (2 * d)):
            a = v[g * 2 * d : g * 2 * d + d]
            b = v[g * 2 * d + d : g * 2 * d + 2 * d]
            hi = jnp.maximum(a, b)
            lo = jnp.minimum(a, b)
            pieces[2 * g], pieces[2 * g + 1] = _dirsel(m, desc, d, hi, lo)
        return jnp.concatenate(pieces, axis=0)
    iota = _iota(v.shape[0])
    bit_j = (iota & d) != 0
    if desc is True or m == 0:
        keep_max = jnp.logical_not(bit_j)
    elif desc is False:
        keep_max = bit_j
    else:
        keep_max = ((iota & m) != 0) == bit_j
    up = pltpu.roll(v, v.shape[0] - d, 0)
    down = pltpu.roll(v, d, 0)
    partner = jnp.where(bit_j, down, up)
    return jnp.where(keep_max, jnp.maximum(v, partner), jnp.minimum(v, partner))


def _apply_run(v, run, cdesc):
    for m, d in run:
        v = _cmpx_chunk(v, m, d, cdesc if m >= _CHUNK else None)
    return v


def _chunk_pass(ref, n, run):
    bigm = sorted({m for m, d in run if m >= _CHUNK})
    nchunks = n // _CHUNK
    if not bigm:

        def body(c, carry):
            base = c * _CHUNK
            ref[pl.ds(base, _CHUNK), :] = _apply_run(
                ref[pl.ds(base, _CHUNK), :], run, None
            )
            return carry

        lax.fori_loop(0, nchunks, body, 0)
        return
    assert len(bigm) == 1
    p = bigm[0] // _CHUNK

    def body(q, carry):
        lo = q & (p - 1)
        c0 = ((q >> p.bit_length() - 1) * 2 * p) | lo
        for c, dv in ((c0, True), (c0 + p, False)):
            base = c * _CHUNK
            ref[pl.ds(base, _CHUNK), :] = _apply_run(
                ref[pl.ds(base, _CHUNK), :], run, dv
            )
        return carry

    lax.fori_loop(0, nchunks // 2, body, 0)


def _pair_stage(ref, n, m, d):
    per_group = d // _CHUNK
    iters = n // (2 * _CHUNK)

    def do(base, desc):
        a = ref[pl.ds(base, _CHUNK), :]
        b = ref[pl.ds(base + d, _CHUNK), :]
        hi = jnp.maximum(a, b)
        lo = jnp.minimum(a, b)
        top, bot = _dirsel(m, desc, _CHUNK, hi, lo)
        ref[pl.ds(base, _CHUNK), :] = top
        ref[pl.ds(base + d, _CHUNK), :] = bot

    if m == 0 or m < _CHUNK:

        def body(it, carry):
            g = it // per_group
            s = it % per_group
            do(g * 2 * d + s * _CHUNK, True if m == 0 else None)
            return carry

        lax.fori_loop(0, iters, body, 0)
        return
    assert m >= 2 * d
    p = m // (2 * d)

    def body(q, carry):
        s = q % per_group
        h = q // per_group
        lo = h & (p - 1)
        g0 = ((h >> p.bit_length() - 1) * 2 * p) | lo
        do(g0 * 2 * d + s * _CHUNK, True)
        do((g0 + p) * 2 * d + s * _CHUNK, False)
        return carry

    lax.fori_loop(0, iters // 2, body, 0)


def _pair_fused(ref, n, m, d):
    sub = d // 2
    per_group = sub // _CHUNK
    iters = n // (4 * _CHUNK)

    def do4(base, desc):
        q0 = ref[pl.ds(base, _CHUNK), :]
        q1 = ref[pl.ds(base + sub, _CHUNK), :]
        q2 = ref[pl.ds(base + d, _CHUNK), :]
        q3 = ref[pl.ds(base + d + sub, _CHUNK), :]
        a0, b2 = _dirsel(m, desc, _CHUNK, jnp.maximum(q0, q2), jnp.minimum(q0, q2))
        a1, b3 = _dirsel(m, desc, _CHUNK, jnp.maximum(q1, q3), jnp.minimum(q1, q3))
        r0, r1 = _dirsel(m, desc, _CHUNK, jnp.maximum(a0, a1), jnp.minimum(a0, a1))
        r2, r3 = _dirsel(m, desc, _CHUNK, jnp.maximum(b2, b3), jnp.minimum(b2, b3))
        ref[pl.ds(base, _CHUNK), :] = r0
        ref[pl.ds(base + sub, _CHUNK), :] = r1
        ref[pl.ds(base + d, _CHUNK), :] = r2
        ref[pl.ds(base + d + sub, _CHUNK), :] = r3

    if m == 0 or m < _CHUNK:

        def body(it, carry):
            g = it // per_group
            s = it % per_group
            do4(g * 2 * d + s * _CHUNK, True if m == 0 else None)
            return carry

        lax.fori_loop(0, iters, body, 0)
        return
    assert m >= 2 * d
    p = m // (2 * d)

    def body(q, carry):
        s = q % per_group
        h = q // per_group
        lo = h & (p - 1)
        g0 = ((h >> p.bit_length() - 1) * 2 * p) | lo
        do4(g0 * 2 * d + s * _CHUNK, True)
        do4((g0 + p) * 2 * d + s * _CHUNK, False)
        return carry

    lax.fori_loop(0, iters // 2, body, 0)


def _run_stages(ref, n, stages):
    run = []
    i = 0
    while i < len(stages):
        m, d = stages[i]
        if d < _CHUNK:
            run.append((m, d))
            i += 1
            continue
        if run:
            _chunk_pass(ref, n, run)
            run = []
        nxt = stages[i + 1] if i + 1 < len(stages) else None
        if nxt is not None and nxt[0] == m and nxt[1] == d // 2 and d // 2 >= _CHUNK:
            _pair_fused(ref, n, m, d)
            i += 2
        else:
            _pair_stage(ref, n, m, d)
            i += 1
    if run:
        _chunk_pass(ref, n, run)


def _topk_body(x_ref, o_ref):
    n = x_ref.shape[0]
    nb = n.bit_length() - 1
    half = n // 2
    pi = _phys_map(nb)

    plan = _plan(nb)
    cut = max(i for i, (m, d) in enumerate(plan) if d >= _CHUNK)
    main, tail = plan[: cut + 1], plan[cut + 1 :]
    merge = [(0, 2 ** pi[a]) for a in range(nb - 2, -1, -1)]
    nhead = 0
    while nhead < len(merge) and merge[nhead][1] < _CHUNK:
        nhead += 1
    mhead, mrest = merge[:nhead], merge[nhead:]

    _run_stages(x_ref, n, main)

    def extract(c, carry):
        base = c * _CHUNK
        va = _apply_run(x_ref[pl.ds(base, _CHUNK), :], tail, True)
        vb = _apply_run(x_ref[pl.ds(base + half, _CHUNK), :], tail, False)
        v = _apply_run(jnp.maximum(va, vb), mhead, True)
        o_ref[pl.ds(base, _CHUNK), :] = v
        return carry

    lax.fori_loop(0, half // _CHUNK, extract, 0)
    _run_stages(o_ref, half, mrest)


def _rev_perm(half, nb):
    pi = _phys_map(nb)
    r = np.arange(half)
    perm = np.zeros(half, dtype=np.int32)
    for bit in range(nb - 1):
        perm |= ((r >> bit) & 1) << pi[bit]
    return perm


def kernel(x):
    b, t, c = x.shape
    rows = b * t
    kk = c // 2
    nb = c.bit_length() - 1
    xt = x.reshape(rows, c).T
    out_t = pl.pallas_call(
        _topk_body,
        grid=(rows // _LANES,),
        in_specs=[pl.BlockSpec((c, _LANES), lambda i: (0, i))],
        out_specs=pl.BlockSpec((kk, _LANES), lambda i: (0, i)),
        out_shape=jax.ShapeDtypeStruct((kk, rows), x.dtype),
        compiler_params=pltpu.CompilerParams(
            dimension_semantics=("parallel",),
        ),
    )(xt)
    out_nat = out_t[jnp.asarray(_rev_perm(kk, nb)), :]
    return out_nat.T.reshape(b, t, kk)


# speedup vs baseline: 1.0822x; 1.0822x over previous
import numpy as np

import jax
import jax.numpy as jnp
from jax import lax
from jax.experimental import pallas as pl
from jax.experimental.pallas import tpu as pltpu

_LANES = 256
_CHUNK = 128


def _phys_map(nb):
    pref = [6, 5, 4, 3] + list(range(7, nb - 1)) + [2, 1, 0]
    pref = [p for p in pref if p <= nb - 2]
    pi = {a: pref[a] for a in range(nb - 1)}
    pi[nb - 1] = nb - 1
    return pi


def _plan(nb):
    pi = _phys_map(nb)
    stages = []
    for b in range(1, nb):
        m = 2 ** pi[b]
        for a in range(b - 1, -1, -1):
            stages.append((m, 2 ** pi[a]))
    return stages


def _neg_transform(stages):
    out = []
    cur = 0
    for m, d in stages:
        tgt = m if 0 < m < 8 else 0
        out.append((0 if tgt else m, d, cur ^ tgt))
        cur = tgt
    assert cur == 0
    return out


def _sign(pre, rows):
    p = _iota(rows) & pre
    parity = (p ^ (p >> 1) ^ (p >> 2)) & 1
    return jnp.where(parity != 0, jnp.float32(-1.0), jnp.float32(1.0))


def _iota(n):
    return jax.lax.broadcasted_iota(jnp.int32, (n, 1), 0)


def _dirsel(m, desc, rows, hi, lo):
    if desc is not None:
        return (hi, lo) if desc else (lo, hi)
    if m == 0:
        return hi, lo
    d = (_iota(hi.shape[0]) & m) == 0
    return jnp.where(d, hi, lo), jnp.where(d, lo, hi)


def _cmpx_chunk(v, m, d, desc):
    if d >= 8:
        pieces = [None] * (2 * (v.shape[0] // (2 * d)))
        for g in range(v.shape[0] // (2 * d)):
            a = v[g * 2 * d : g * 2 * d + d]
            b = v[g * 2 * d + d : g * 2 * d + 2 * d]
            hi = jnp.maximum(a, b)
            lo = jnp.minimum(a, b)
            pieces[2 * g], pieces[2 * g + 1] = _dirsel(m, desc, d, hi, lo)
        return jnp.concatenate(pieces, axis=0)
    iota = _iota(v.shape[0])
    bit_j = (iota & d) != 0
    if desc is True or m == 0:
        keep_max = jnp.logical_not(bit_j)
    elif desc is False:
        keep_max = bit_j
    else:
        keep_max = ((iota & m) != 0) == bit_j
    up = pltpu.roll(v, v.shape[0] - d, 0)
    down = pltpu.roll(v, d, 0)
    partner = jnp.where(bit_j, down, up)
    return jnp.where(keep_max, jnp.maximum(v, partner), jnp.minimum(v, partner))


def _apply_run(v, run, cdesc):
    for m, d, pre in run:
        if pre:
            v = v * _sign(pre, v.shape[0])
        v = _cmpx_chunk(v, m, d, cdesc if m >= _CHUNK else None)
    return v


def _chunk_pass(ref, n, run):
    bigm = sorted({m for m, d, pre in run if m >= _CHUNK})
    nchunks = n // _CHUNK
    if not bigm:

        def body(c, carry):
            base = c * _CHUNK
            ref[pl.ds(base, _CHUNK), :] = _apply_run(
                ref[pl.ds(base, _CHUNK), :], run, None
            )
            return carry

        lax.fori_loop(0, nchunks, body, 0)
        return
    assert len(bigm) == 1
    p = bigm[0] // _CHUNK

    def body(q, carry):
        lo = q & (p - 1)
        c0 = ((q >> p.bit_length() - 1) * 2 * p) | lo
        for c, dv in ((c0, True), (c0 + p, False)):
            base = c * _CHUNK
            ref[pl.ds(base, _CHUNK), :] = _apply_run(
                ref[pl.ds(base, _CHUNK), :], run, dv
            )
        return carry

    lax.fori_loop(0, nchunks // 2, body, 0)


def _pair_stage(ref, n, m, d, pre):
    per_group = d // _CHUNK
    iters = n // (2 * _CHUNK)

    def do(base, desc):
        a = ref[pl.ds(base, _CHUNK), :]
        b = ref[pl.ds(base + d, _CHUNK), :]
        if pre:
            s = _sign(pre, _CHUNK)
            a = a * s
            b = b * s
        hi = jnp.maximum(a, b)
        lo = jnp.minimum(a, b)
        top, bot = _dirsel(m, desc, _CHUNK, hi, lo)
        ref[pl.ds(base, _CHUNK), :] = top
        ref[pl.ds(base + d, _CHUNK), :] = bot

    if m == 0 or m < _CHUNK:

        def body(it, carry):
            g = it // per_group
            s = it % per_group
            do(g * 2 * d + s * _CHUNK, True if m == 0 else None)
            return carry

        lax.fori_loop(0, iters, body, 0)
        return
    assert m >= 2 * d
    p = m // (2 * d)

    def body(q, carry):
        s = q % per_group
        h = q // per_group
        lo = h & (p - 1)
        g0 = ((h >> p.bit_length() - 1) * 2 * p) | lo
        do(g0 * 2 * d + s * _CHUNK, True)
        do((g0 + p) * 2 * d + s * _CHUNK, False)
        return carry

    lax.fori_loop(0, iters // 2, body, 0)


def _pair_fused(ref, n, m, d, pre):
    sub = d // 2
    per_group = sub // _CHUNK
    iters = n // (4 * _CHUNK)

    def do4(base, desc):
        q0 = ref[pl.ds(base, _CHUNK), :]
        q1 = ref[pl.ds(base + sub, _CHUNK), :]
        q2 = ref[pl.ds(base + d, _CHUNK), :]
        q3 = ref[pl.ds(base + d + sub, _CHUNK), :]
        if pre:
            s = _sign(pre, _CHUNK)
            q0 = q0 * s
            q1 = q1 * s
            q2 = q2 * s
            q3 = q3 * s
        a0, b2 = _dirsel(m, desc, _CHUNK, jnp.maximum(q0, q2), jnp.minimum(q0, q2))
        a1, b3 = _dirsel(m, desc, _CHUNK, jnp.maximum(q1, q3), jnp.minimum(q1, q3))
        r0, r1 = _dirsel(m, desc, _CHUNK, jnp.maximum(a0, a1), jnp.minimum(a0, a1))
        r2, r3 = _dirsel(m, desc, _CHUNK, jnp.maximum(b2, b3), jnp.minimum(b2, b3))
        ref[pl.ds(base, _CHUNK), :] = r0
        ref[pl.ds(base + sub, _CHUNK), :] = r1
        ref[pl.ds(base + d, _CHUNK), :] = r2
        ref[pl.ds(base + d + sub, _CHUNK), :] = r3

    if m == 0 or m < _CHUNK:

        def body(it, carry):
            g = it // per_group
            s = it % per_group
            do4(g * 2 * d + s * _CHUNK, True if m == 0 else None)
            return carry

        lax.fori_loop(0, iters, body, 0)
        return
    assert m >= 2 * d
    p = m // (2 * d)

    def body(q, carry):
        s = q % per_group
        h = q // per_group
        lo = h & (p - 1)
        g0 = ((h >> p.bit_length() - 1) * 2 * p) | lo
        do4(g0 * 2 * d + s * _CHUNK, True)
        do4((g0 + p) * 2 * d + s * _CHUNK, False)
        return carry

    lax.fori_loop(0, iters // 2, body, 0)


def _run_stages(ref, n, stages):
    run = []
    i = 0
    while i < len(stages):
        m, d, pre = stages[i]
        if d < _CHUNK:
            run.append((m, d, pre))
            i += 1
            continue
        if run:
            _chunk_pass(ref, n, run)
            run = []
        nxt = stages[i + 1] if i + 1 < len(stages) else None
        if (
            nxt is not None
            and nxt[0] == m
            and nxt[1] == d // 2
            and d // 2 >= _CHUNK
            and nxt[2] == 0
        ):
            _pair_fused(ref, n, m, d, pre)
            i += 2
        else:
            _pair_stage(ref, n, m, d, pre)
            i += 1
    if run:
        _chunk_pass(ref, n, run)


def _topk_body(x_ref, o_ref):
    n = x_ref.shape[0]
    nb = n.bit_length() - 1
    half = n // 2
    pi = _phys_map(nb)

    plan = _neg_transform(_plan(nb))
    cut = max(i for i, (m, d, pre) in enumerate(plan) if d >= _CHUNK)
    main, tail = plan[: cut + 1], plan[cut + 1 :]
    merge = [(0, 2 ** pi[a], 0) for a in range(nb - 2, -1, -1)]
    nhead = 0
    while nhead < len(merge) and merge[nhead][1] < _CHUNK:
        nhead += 1
    mhead, mrest = merge[:nhead], merge[nhead:]

    _run_stages(x_ref, n, main)

    def extract(c, carry):
        base = c * _CHUNK
        va = _apply_run(x_ref[pl.ds(base, _CHUNK), :], tail, True)
        vb = _apply_run(x_ref[pl.ds(base + half, _CHUNK), :], tail, False)
        v = _apply_run(jnp.maximum(va, vb), mhead, True)
        o_ref[pl.ds(base, _CHUNK), :] = v
        return carry

    lax.fori_loop(0, half // _CHUNK, extract, 0)
    _run_stages(o_ref, half, mrest)


def _rev_perm(half, nb):
    pi = _phys_map(nb)
    r = np.arange(half)
    perm = np.zeros(half, dtype=np.int32)
    for bit in range(nb - 1):
        perm |= ((r >> bit) & 1) << pi[bit]
    return perm


def kernel(x):
    b, t, c = x.shape
    rows = b * t
    kk = c // 2
    nb = c.bit_length() - 1
    xt = x.reshape(rows, c).T
    out_t = pl.pallas_call(
        _topk_body,
        grid=(rows // _LANES,),
        in_specs=[pl.BlockSpec((c, _LANES), lambda i: (0, i))],
        out_specs=pl.BlockSpec((kk, _LANES), lambda i: (0, i)),
        out_shape=jax.ShapeDtypeStruct((kk, rows), x.dtype),
        compiler_params=pltpu.CompilerParams(
            dimension_semantics=("parallel",),
        ),
    )(xt)
    out_nat = out_t[jnp.asarray(_rev_perm(kk, nb)), :]
    return out_nat.T.reshape(b, t, kk)
